# Initial kernel scaffold; baseline (speedup 1.0000x reference)
import functools

import jax
import jax.numpy as jnp
from jax import lax
from jax.experimental import pallas as pl
from jax.experimental.pallas import tpu as pltpu
from jax.experimental.pallas import tpu_sc as plsc

VOCAB = 1000000
D = 32
B = 16384
L = 200

VC = 64
VR = VOCAB // VC
VBM = 125
VGRID = VR // VBM

NC = 2
NS = 16
NW = NC * NS
TOTAL_IDX = B * L
PER_W = TOTAL_IDX // NW
SUB = 128
NSUB = 8
CHUNK = SUB * NSUB
ITERS = PER_W // CHUNK

PBB = 1024
PGRID = B // PBB


def _project_body(emb_ref, w_ref, v_ref):
    w = w_ref[0]
    v_ref[...] = jnp.sum(emb_ref[...] * w[None, None, :], axis=-1)


@jax.jit
def _project(emb3, fc_w):
    return pl.pallas_call(
        _project_body,
        grid=(VGRID,),
        in_specs=[
            pl.BlockSpec((VBM, VC, D), lambda i: (i, 0, 0)),
            pl.BlockSpec((1, D), lambda i: (0, 0)),
        ],
        out_specs=pl.BlockSpec((VBM, VC), lambda i: (i, 0)),
        out_shape=jax.ShapeDtypeStruct((VR, VC), jnp.float32),
    )(emb3, fc_w)


def _gather_body(x_hbm, v_hbm, out_hbm, idx_v, val_v, sem):
    wid = lax.axis_index("s") * NC + lax.axis_index("c")
    base = wid * PER_W

    def body(i, carry):
        off = base + i * CHUNK
        pltpu.sync_copy(x_hbm.at[pl.ds(off, CHUNK)], idx_v)
        copies = [
            pltpu.make_async_copy(
                v_hbm.at[idx_v.at[pl.ds(j * SUB, SUB)]],
                val_v.at[pl.ds(j * SUB, SUB)],
                sem,
            )
            for j in range(NSUB)
        ]
        for c in copies:
            c.start()
        for c in copies:
            c.wait()
        pltpu.sync_copy(val_v, out_hbm.at[pl.ds(off, CHUNK)])
        return carry

    lax.fori_loop(0, ITERS, body, 0)


@jax.jit
def _gather(x_flat, v):
    mesh = plsc.VectorSubcoreMesh(
        core_axis_name="c", subcore_axis_name="s", num_cores=NC, num_subcores=NS
    )
    return pl.kernel(
        _gather_body,
        out_type=jax.ShapeDtypeStruct((TOTAL_IDX,), jnp.float32),
        mesh=mesh,
        scratch_types=[
            pltpu.VMEM((CHUNK,), jnp.int32),
            pltpu.VMEM((CHUNK,), jnp.float32),
            pltpu.SemaphoreType.DMA,
        ],
    )(x_flat, v)


def _pool_body(g_ref, len_ref, b_ref, o_ref):
    pos = lax.broadcasted_iota(jnp.int32, (PBB, L), 1)
    lens = len_ref[...]
    masked = jnp.where(pos < lens, g_ref[...], 0.0)
    s = jnp.sum(masked, axis=1, keepdims=True)
    o_ref[...] = s / lens.astype(jnp.float32) + b_ref[0, 0]


@jax.jit
def _pool(g2, len2, fc_b2):
    return pl.pallas_call(
        _pool_body,
        grid=(PGRID,),
        in_specs=[
            pl.BlockSpec((PBB, L), lambda i: (i, 0)),
            pl.BlockSpec((PBB, 1), lambda i: (i, 0)),
            pl.BlockSpec((1, 1), lambda i: (0, 0)),
        ],
        out_specs=pl.BlockSpec((PBB, 1), lambda i: (i, 0)),
        out_shape=jax.ShapeDtypeStruct((B, 1), jnp.float32),
    )(g2, len2, fc_b2)


def kernel(x, lengths, emb_table, fc_w, fc_b):
    x_flat = x.reshape(TOTAL_IDX).astype(jnp.int32)
    emb3 = emb_table.reshape(VR, VC, D)
    v = _project(emb3, fc_w).reshape(VOCAB)
    g = _gather(x_flat, v)
    out = _pool(
        g.reshape(B, L),
        lengths.reshape(B, 1).astype(jnp.int32),
        fc_b.reshape(1, 1),
    )
    return out.reshape(B)


# speedup vs baseline: 16.2005x; 16.2005x over previous
import functools

import jax
import jax.numpy as jnp
from jax import lax
from jax.experimental import pallas as pl
from jax.experimental.pallas import tpu as pltpu
from jax.experimental.pallas import tpu_sc as plsc

VOCAB = 1000000
D = 32
B = 16384
L = 200

VC = 64
VR = VOCAB // VC
VBM = 125
VGRID = VR // VBM

NC = 2
NS = 16
NW = NC * NS
TOTAL_IDX = B * L
PER_W = TOTAL_IDX // NW
SUB = 128
NSUB = 8
CHUNK = SUB * NSUB
ITERS = PER_W // CHUNK

PBB = 1024
PGRID = B // PBB


def _project_body(emb_ref, w_ref, v_ref):
    w = w_ref[0]
    v_ref[...] = jnp.sum(emb_ref[...] * w[None, None, :], axis=-1)[None]


@jax.jit
def _project(emb3, fc_w):
    return pl.pallas_call(
        _project_body,
        grid=(VGRID,),
        in_specs=[
            pl.BlockSpec((VBM, VC, D), lambda i: (i, 0, 0)),
            pl.BlockSpec((1, D), lambda i: (0, 0)),
        ],
        out_specs=pl.BlockSpec((1, VBM, VC), lambda i: (i, 0, 0)),
        out_shape=jax.ShapeDtypeStruct((VGRID, VBM, VC), jnp.float32),
    )(emb3, fc_w)


def _gather_body(x_hbm, v_hbm, out_hbm, idx_v, val_v, sem):
    wid = lax.axis_index("s") * NC + lax.axis_index("c")
    base = wid * PER_W

    def body(i, carry):
        off = base + i * CHUNK
        pltpu.sync_copy(x_hbm.at[pl.ds(off, CHUNK)], idx_v)
        copies = [
            pltpu.make_async_copy(
                v_hbm.at[idx_v.at[pl.ds(j * SUB, SUB)]],
                val_v.at[pl.ds(j * SUB, SUB)],
                sem,
            )
            for j in range(NSUB)
        ]
        for c in copies:
            c.start()
        for c in copies:
            c.wait()
        pltpu.sync_copy(val_v, out_hbm.at[pl.ds(off, CHUNK)])
        return carry

    lax.fori_loop(0, ITERS, body, 0)


@jax.jit
def _gather(x_flat, v):
    mesh = plsc.VectorSubcoreMesh(
        core_axis_name="c", subcore_axis_name="s", num_cores=NC, num_subcores=NS
    )
    return pl.kernel(
        _gather_body,
        out_type=jax.ShapeDtypeStruct((TOTAL_IDX,), jnp.float32),
        mesh=mesh,
        scratch_types=[
            pltpu.VMEM((CHUNK,), jnp.int32),
            pltpu.VMEM((CHUNK,), jnp.float32),
            pltpu.SemaphoreType.DMA,
        ],
    )(x_flat, v)


def _pool_body(g_ref, len_ref, b_ref, o_ref):
    pos = lax.broadcasted_iota(jnp.int32, (PBB, L), 1)
    lens = len_ref[...]
    masked = jnp.where(pos < lens, g_ref[...], 0.0)
    s = jnp.sum(masked, axis=1, keepdims=True)
    o_ref[...] = s / lens.astype(jnp.float32) + b_ref[0, 0]


@jax.jit
def _pool(g2, len2, fc_b2):
    return pl.pallas_call(
        _pool_body,
        grid=(PGRID,),
        in_specs=[
            pl.BlockSpec((PBB, L), lambda i: (i, 0)),
            pl.BlockSpec((PBB, 1), lambda i: (i, 0)),
            pl.BlockSpec((1, 1), lambda i: (0, 0)),
        ],
        out_specs=pl.BlockSpec((PBB, 1), lambda i: (i, 0)),
        out_shape=jax.ShapeDtypeStruct((B, 1), jnp.float32),
    )(g2, len2, fc_b2)


def kernel(x, lengths, emb_table, fc_w, fc_b):
    x_flat = x.reshape(TOTAL_IDX).astype(jnp.int32)
    emb3 = emb_table.reshape(VR, VC, D)
    v = _project(emb3, fc_w).reshape(VOCAB)
    g = _gather(x_flat, v)
    out = _pool(
        g.reshape(B, L),
        lengths.reshape(B, 1).astype(jnp.int32),
        fc_b.reshape(1, 1),
    )
    return out.reshape(B)


# speedup vs baseline: 16.2123x; 1.0007x over previous
import functools

import jax
import jax.numpy as jnp
from jax import lax
from jax.experimental import pallas as pl
from jax.experimental.pallas import tpu as pltpu
from jax.experimental.pallas import tpu_sc as plsc

VOCAB = 1000000
D = 32
B = 16384
L = 200

VC = 64
VR = VOCAB // VC
VBM = 125
VGRID = VR // VBM

NC = 2
NS = 16
NW = NC * NS
TOTAL_IDX = B * L
SUB = 128
XR = TOTAL_IDX // SUB
ROWS_W = XR // NW
NSUB = 8
ITERS = ROWS_W // NSUB

PBB = 1024
PGRID = B // PBB


def _project_body(emb_ref, w_ref, v_ref):
    w = w_ref[0]
    v_ref[...] = jnp.sum(emb_ref[...] * w[None, None, :], axis=-1)[None]


@jax.jit
def _project(emb3, fc_w):
    return pl.pallas_call(
        _project_body,
        grid=(VGRID,),
        in_specs=[
            pl.BlockSpec((VBM, VC, D), lambda i: (i, 0, 0)),
            pl.BlockSpec((1, D), lambda i: (0, 0)),
        ],
        out_specs=pl.BlockSpec((1, VBM, VC), lambda i: (i, 0, 0)),
        out_shape=jax.ShapeDtypeStruct((VGRID, VBM, VC), jnp.float32),
    )(emb3, fc_w)


def _gather_body(x_hbm, v_hbm, out_hbm, idx_v, val_v, sem):
    wid = lax.axis_index("s") * NC + lax.axis_index("c")
    base = wid * ROWS_W

    def body(i, carry):
        off = base + i * NSUB
        pltpu.sync_copy(x_hbm.at[pl.ds(off, NSUB)], idx_v)
        copies = [
            pltpu.make_async_copy(
                v_hbm.at[idx_v.at[j]],
                val_v.at[j],
                sem,
            )
            for j in range(NSUB)
        ]
        for c in copies:
            c.start()
        for c in copies:
            c.wait()
        pltpu.sync_copy(val_v, out_hbm.at[pl.ds(off, NSUB)])
        return carry

    lax.fori_loop(0, ITERS, body, 0)


@jax.jit
def _gather(x2, v):
    mesh = plsc.VectorSubcoreMesh(
        core_axis_name="c", subcore_axis_name="s", num_cores=NC, num_subcores=NS
    )
    return pl.kernel(
        _gather_body,
        out_type=jax.ShapeDtypeStruct((XR, SUB), jnp.float32),
        mesh=mesh,
        scratch_types=[
            pltpu.VMEM((NSUB, SUB), jnp.int32),
            pltpu.VMEM((NSUB, SUB), jnp.float32),
            pltpu.SemaphoreType.DMA,
        ],
    )(x2, v)


def _pool_body(g_ref, len_ref, b_ref, o_ref):
    pos = lax.broadcasted_iota(jnp.int32, (PBB, L), 1)
    lens = len_ref[...]
    masked = jnp.where(pos < lens, g_ref[...], 0.0)
    s = jnp.sum(masked, axis=1, keepdims=True)
    o_ref[...] = s / lens.astype(jnp.float32) + b_ref[0, 0]


@jax.jit
def _pool(g2, len2, fc_b2):
    return pl.pallas_call(
        _pool_body,
        grid=(PGRID,),
        in_specs=[
            pl.BlockSpec((PBB, L), lambda i: (i, 0)),
            pl.BlockSpec((PBB, 1), lambda i: (i, 0)),
            pl.BlockSpec((1, 1), lambda i: (0, 0)),
        ],
        out_specs=pl.BlockSpec((PBB, 1), lambda i: (i, 0)),
        out_shape=jax.ShapeDtypeStruct((B, 1), jnp.float32),
    )(g2, len2, fc_b2)


def kernel(x, lengths, emb_table, fc_w, fc_b):
    x2 = x.reshape(XR, SUB).astype(jnp.int32)
    emb3 = emb_table.reshape(VR, VC, D)
    v = _project(emb3, fc_w).reshape(VOCAB)
    g = _gather(x2, v)
    out = _pool(
        g.reshape(B, L),
        lengths.reshape(B, 1).astype(jnp.int32),
        fc_b.reshape(1, 1),
    )
    return out.reshape(B)


# speedup vs baseline: 16.2167x; 1.0003x over previous
import functools

import jax
import jax.numpy as jnp
from jax import lax
from jax.experimental import pallas as pl
from jax.experimental.pallas import tpu as pltpu
from jax.experimental.pallas import tpu_sc as plsc

VOCAB = 1000000
D = 32
B = 16384
L = 200

VC = 64
VR = VOCAB // VC
VBM = 125
VGRID = VR // VBM

NC = 2
NS = 16
NW = NC * NS
TOTAL_IDX = B * L
SUB = 128
XR = TOTAL_IDX // SUB
ROWS_W = XR // NW
NSUB = 8
ITERS = ROWS_W // NSUB

PBB = 1024
PGRID = B // PBB


def _project_body(emb_ref, w_ref, v_ref):
    w = w_ref[0]
    v_ref[...] = jnp.sum(emb_ref[...] * w[None, None, :], axis=-1)[None]


@jax.jit
def _project(emb3, fc_w):
    return pl.pallas_call(
        _project_body,
        grid=(VGRID,),
        in_specs=[
            pl.BlockSpec((VBM, VC, D), lambda i: (i, 0, 0)),
            pl.BlockSpec((1, D), lambda i: (0, 0)),
        ],
        out_specs=pl.BlockSpec((1, VBM, VC), lambda i: (i, 0, 0)),
        out_shape=jax.ShapeDtypeStruct((VGRID, VBM, VC), jnp.float32),
    )(emb3, fc_w)


def _gather_body(x_hbm, v_hbm, out_hbm, idx_v, val_v, sem):
    wid = lax.axis_index("s") * NC + lax.axis_index("c")
    base = wid * ROWS_W

    def body(i, carry):
        off = base + i * NSUB
        pltpu.sync_copy(x_hbm.at[pl.ds(off, NSUB)], idx_v)
        copies = [
            pltpu.make_async_copy(
                v_hbm.at[idx_v.at[j]],
                val_v.at[j],
                sem,
            )
            for j in range(NSUB)
        ]
        for c in copies:
            c.start()
        for c in copies:
            c.wait()
        pltpu.sync_copy(val_v, out_hbm.at[pl.ds(off, NSUB)])
        return carry

    lax.fori_loop(0, ITERS, body, 0)


@jax.jit
def _gather(x2, v):
    mesh = plsc.VectorSubcoreMesh(
        core_axis_name="c", subcore_axis_name="s", num_cores=NC, num_subcores=NS
    )
    return pl.kernel(
        _gather_body,
        out_type=jax.ShapeDtypeStruct((XR, SUB), jnp.float32),
        mesh=mesh,
        scratch_types=[
            pltpu.VMEM((NSUB, SUB), jnp.int32),
            pltpu.VMEM((NSUB, SUB), jnp.float32),
            pltpu.SemaphoreType.DMA,
        ],
    )(x2, v)


def _pool_body(g_ref, len_ref, b_ref, o_ref):
    pos = lax.broadcasted_iota(jnp.int32, (PBB, L), 1)
    lens = len_ref[...]
    masked = jnp.where(pos < lens, g_ref[...], 0.0)
    s = jnp.sum(masked, axis=1, keepdims=True)
    o_ref[...] = s / lens.astype(jnp.float32) + b_ref[0, 0]


@jax.jit
def _pool(g2, len2, fc_b2):
    return pl.pallas_call(
        _pool_body,
        grid=(PGRID,),
        in_specs=[
            pl.BlockSpec((PBB, L), lambda i: (i, 0)),
            pl.BlockSpec((PBB, 1), lambda i: (i, 0)),
            pl.BlockSpec((1, 1), lambda i: (0, 0)),
        ],
        out_specs=pl.BlockSpec((PBB, 1), lambda i: (i, 0)),
        out_shape=jax.ShapeDtypeStruct((B, 1), jnp.float32),
    )(g2, len2, fc_b2)


def kernel(x, lengths, emb_table, fc_w, fc_b):
    emb3 = emb_table.reshape(VR, VC, D)
    v = _project(emb3, fc_w).reshape(VOCAB)
    x2 = x.reshape(XR, SUB).astype(jnp.int32)
    g = _gather(x2, v)
    out = _pool(
        g.reshape(B, L),
        lengths.reshape(B, 1).astype(jnp.int32),
        fc_b.reshape(1, 1),
    )
    return out.reshape(B)
